# baseline (speedup 1.0000x reference)
import functools

import jax
import jax.numpy as jnp
from jax.experimental import pallas as pl
from jax.experimental.pallas import tpu as pltpu


def _assign_kernel(p_ref, c_ref, o_ref, pt_ref, m_ref, i_ref, *, kc, nsteps):
    step = pl.program_id(0)

    @pl.when(step == 0)
    def _prep():
        pt_ref[...] = p_ref[...].T

    c = c_ref[...]
    cnorm = jnp.sum(c * c, axis=1, keepdims=True)
    scores = cnorm - 2.0 * jax.lax.dot_general(
        c, pt_ref[...],
        dimension_numbers=(((1,), (0,)), ((), ())),
        preferred_element_type=jnp.float32,
        precision=jax.lax.Precision.HIGHEST,
    )
    m = jnp.min(scores, axis=0, keepdims=True)
    iota = jax.lax.broadcasted_iota(jnp.int32, scores.shape, 0)
    idx = jnp.min(jnp.where(scores == m, iota, kc), axis=0,
                  keepdims=True) + step * kc

    @pl.when(step == 0)
    def _init():
        m_ref[...] = m
        i_ref[...] = idx

    @pl.when(step > 0)
    def _merge():
        better = m < m_ref[...]
        i_ref[...] = jnp.where(better, idx, i_ref[...])
        m_ref[...] = jnp.minimum(m, m_ref[...])

    @pl.when(step == nsteps - 1)
    def _out():
        o_ref[...] = i_ref[...]


def kernel(points, centroids):
    b, d = points.shape
    k = centroids.shape[0]
    kc = 128
    nsteps = k // kc
    body = functools.partial(_assign_kernel, kc=kc, nsteps=nsteps)
    out = pl.pallas_call(
        body,
        grid=(nsteps,),
        in_specs=[
            pl.BlockSpec((b, d), lambda i: (0, 0)),
            pl.BlockSpec((kc, d), lambda i: (i, 0)),
        ],
        out_specs=pl.BlockSpec((1, b), lambda i: (0, 0)),
        out_shape=jax.ShapeDtypeStruct((1, b), jnp.int32),
        scratch_shapes=[
            pltpu.VMEM((d, b), jnp.float32),
            pltpu.VMEM((1, b), jnp.float32),
            pltpu.VMEM((1, b), jnp.int32),
        ],
    )(points, centroids)
    return out.reshape(b)


# speedup vs baseline: 1.2444x; 1.2444x over previous
import functools

import jax
import jax.numpy as jnp
from jax.experimental import pallas as pl
from jax.experimental.pallas import tpu as pltpu


def _assign_kernel(p_ref, c_ref, o_ref, pt_ref, m_ref, i_ref, *, kc, nsteps):
    step = pl.program_id(0)

    @pl.when(step == 0)
    def _prep():
        pt_ref[...] = p_ref[...].T

    c = c_ref[...]
    cnorm = jnp.sum(c * c, axis=1, keepdims=True)
    scores = cnorm - 2.0 * jax.lax.dot_general(
        c, pt_ref[...],
        dimension_numbers=(((1,), (0,)), ((), ())),
        preferred_element_type=jnp.float32,
        precision=jax.lax.Precision.HIGHEST,
    )
    m = jnp.min(scores, axis=0, keepdims=True)
    iota = jax.lax.broadcasted_iota(jnp.int32, scores.shape, 0)
    idx = jnp.min(jnp.where(scores == m, iota, kc), axis=0,
                  keepdims=True) + step * kc

    @pl.when(step == 0)
    def _init():
        m_ref[...] = m
        i_ref[...] = idx

    @pl.when(step > 0)
    def _merge():
        better = m < m_ref[...]
        i_ref[...] = jnp.where(better, idx, i_ref[...])
        m_ref[...] = jnp.minimum(m, m_ref[...])

    @pl.when(step == nsteps - 1)
    def _out():
        o_ref[...] = i_ref[...]


def kernel(points, centroids):
    b, d = points.shape
    k = centroids.shape[0]
    kc = 512
    nsteps = k // kc
    body = functools.partial(_assign_kernel, kc=kc, nsteps=nsteps)
    out = pl.pallas_call(
        body,
        grid=(nsteps,),
        in_specs=[
            pl.BlockSpec((b, d), lambda i: (0, 0)),
            pl.BlockSpec((kc, d), lambda i: (i, 0)),
        ],
        out_specs=pl.BlockSpec((1, b), lambda i: (0, 0)),
        out_shape=jax.ShapeDtypeStruct((1, b), jnp.int32),
        scratch_shapes=[
            pltpu.VMEM((d, b), jnp.float32),
            pltpu.VMEM((1, b), jnp.float32),
            pltpu.VMEM((1, b), jnp.int32),
        ],
    )(points, centroids)
    return out.reshape(b)


# speedup vs baseline: 1.3451x; 1.0810x over previous
import functools

import jax
import jax.numpy as jnp
from jax.experimental import pallas as pl
from jax.experimental.pallas import tpu as pltpu


def _assign_kernel(p_ref, c_ref, o_ref, pt_ref, m_ref, i_ref, *, kc, nsteps):
    step = pl.program_id(0)

    @pl.when(step == 0)
    def _prep():
        pt_ref[...] = p_ref[...].T

    sub = kc // 2
    m = None
    idx = None
    for s in range(2):
        c = c_ref[pl.ds(s * sub, sub), :]
        cnorm = jnp.sum(c * c, axis=1, keepdims=True)
        scores = cnorm - 2.0 * jax.lax.dot_general(
            c, pt_ref[...],
            dimension_numbers=(((1,), (0,)), ((), ())),
            preferred_element_type=jnp.float32,
            precision=jax.lax.Precision.HIGHEST,
        )
        m_s = jnp.min(scores, axis=0, keepdims=True)
        iota = jax.lax.broadcasted_iota(jnp.int32, scores.shape, 0)
        idx_s = jnp.min(jnp.where(scores == m_s, iota, sub), axis=0,
                        keepdims=True) + (step * kc + s * sub)
        if m is None:
            m, idx = m_s, idx_s
        else:
            better = m_s < m
            idx = jnp.where(better, idx_s, idx)
            m = jnp.minimum(m_s, m)

    @pl.when(step == 0)
    def _init():
        m_ref[...] = m
        i_ref[...] = idx

    @pl.when(step > 0)
    def _merge():
        better = m < m_ref[...]
        i_ref[...] = jnp.where(better, idx, i_ref[...])
        m_ref[...] = jnp.minimum(m, m_ref[...])

    @pl.when(step == nsteps - 1)
    def _out():
        o_ref[...] = i_ref[...]


def kernel(points, centroids):
    b, d = points.shape
    k = centroids.shape[0]
    kc = 512
    nsteps = k // kc
    body = functools.partial(_assign_kernel, kc=kc, nsteps=nsteps)
    out = pl.pallas_call(
        body,
        grid=(nsteps,),
        in_specs=[
            pl.BlockSpec((b, d), lambda i: (0, 0)),
            pl.BlockSpec((kc, d), lambda i: (i, 0)),
        ],
        out_specs=pl.BlockSpec((1, b), lambda i: (0, 0)),
        out_shape=jax.ShapeDtypeStruct((1, b), jnp.int32),
        scratch_shapes=[
            pltpu.VMEM((d, b), jnp.float32),
            pltpu.VMEM((1, b), jnp.float32),
            pltpu.VMEM((1, b), jnp.int32),
        ],
    )(points, centroids)
    return out.reshape(b)
